# baseline (device time: 24098 ns/iter reference)
import jax
import jax.numpy as jnp
from jax import lax
from jax.experimental import pallas as pl
from jax.experimental.pallas import tpu as pltpu

N_DEV = 8
EARLY = 2


def kernel(x, w_mat):
    m_per, k = x.shape
    n = w_mat.shape[1]
    n_per = n // N_DEV

    def body(x_ref, w_hbm, out_ref, wfull, wbuf, ybuf, rbuf,
             wsems, bulk_sem, send_sems, recv_sems):
        my = lax.axis_index("i")

        for m in range(N_DEV):
            @pl.when(my == m)
            def _early_dma():
                for c in range(EARLY):
                    j = (m + 1 + c) % N_DEV
                    pltpu.make_async_copy(
                        w_hbm.at[:, pl.ds(j * n_per, n_per)],
                        wbuf.at[c],
                        wsems.at[c],
                    ).start()

        bulk = pltpu.make_async_copy(w_hbm, wfull, bulk_sem)
        bulk.start()

        barrier_sem = pltpu.get_barrier_semaphore()
        for d in range(N_DEV):
            pl.semaphore_signal(
                barrier_sem, inc=1,
                device_id=(d,), device_id_type=pl.DeviceIdType.MESH,
            )

        for m in range(N_DEV):
            @pl.when(my == m)
            def _dev():
                for c in range(N_DEV):
                    j = (m + 1 + c) % N_DEV
                    if c < EARLY:
                        pltpu.make_async_copy(
                            w_hbm.at[:, pl.ds(j * n_per, n_per)],
                            wbuf.at[c],
                            wsems.at[c],
                        ).wait()
                        w_src = wbuf[c]
                    else:
                        if c == EARLY:
                            bulk.wait()
                        w_src = wfull[:, j * n_per:(j + 1) * n_per]
                    y_c = jnp.maximum(
                        jnp.dot(x_ref[...], w_src,
                                preferred_element_type=jnp.float32),
                        0.0,
                    )
                    if j == m:
                        out_ref[m * m_per:(m + 1) * m_per, :] = y_c
                    else:
                        ybuf[c] = y_c.astype(jnp.bfloat16)
                        if c == 0:
                            pl.semaphore_wait(barrier_sem, N_DEV)
                        pltpu.make_async_remote_copy(
                            src_ref=ybuf.at[c],
                            dst_ref=rbuf.at[m],
                            send_sem=send_sems.at[c],
                            recv_sem=recv_sems.at[m],
                            device_id=(j,),
                            device_id_type=pl.DeviceIdType.MESH,
                        ).start()
                for c in range(N_DEV - 1):
                    src = (m - 1 - c) % N_DEV
                    pltpu.make_async_remote_copy(
                        src_ref=ybuf.at[0],
                        dst_ref=rbuf.at[src],
                        send_sem=send_sems.at[0],
                        recv_sem=recv_sems.at[src],
                        device_id=(src,),
                        device_id_type=pl.DeviceIdType.MESH,
                    ).wait_recv()
                    out_ref[src * m_per:(src + 1) * m_per, :] = (
                        rbuf[src].astype(jnp.float32))

        for c in range(N_DEV - 1):
            pltpu.make_async_remote_copy(
                src_ref=ybuf.at[c],
                dst_ref=rbuf.at[0],
                send_sem=send_sems.at[c],
                recv_sem=recv_sems.at[0],
                device_id=(0,),
                device_id_type=pl.DeviceIdType.MESH,
            ).wait_send()

    return pl.pallas_call(
        body,
        out_shape=jax.ShapeDtypeStruct((N_DEV * m_per, n_per), jnp.float32),
        in_specs=[
            pl.BlockSpec(memory_space=pltpu.MemorySpace.VMEM),
            pl.BlockSpec(memory_space=pl.ANY),
        ],
        out_specs=pl.BlockSpec(memory_space=pltpu.MemorySpace.VMEM),
        scratch_shapes=[
            pltpu.VMEM((k, n), jnp.float32),
            pltpu.VMEM((EARLY, k, n_per), jnp.float32),
            pltpu.VMEM((N_DEV - 1, m_per, n_per), jnp.bfloat16),
            pltpu.VMEM((N_DEV, m_per, n_per), jnp.bfloat16),
            pltpu.SemaphoreType.DMA((EARLY,)),
            pltpu.SemaphoreType.DMA,
            pltpu.SemaphoreType.DMA((N_DEV - 1,)),
            pltpu.SemaphoreType.DMA((N_DEV,)),
        ],
        compiler_params=pltpu.CompilerParams(collective_id=0),
    )(x, w_mat)


# device time: 20786 ns/iter; 1.1593x vs baseline; 1.1593x over previous
import jax
import jax.numpy as jnp
from jax import lax
from jax.experimental import pallas as pl
from jax.experimental.pallas import tpu as pltpu

N_DEV = 8


def kernel(x, w_mat):
    m_per, k = x.shape
    n = w_mat.shape[1]
    n_per = n // N_DEV

    def body(x_ref, w_ref, out_ref, ybuf, rbuf, send_sems, recv_sems,
             entry_sems):
        my = lax.axis_index("i")

        barrier_sem = pltpu.get_barrier_semaphore()
        pl.semaphore_signal(barrier_sem, inc=1)
        pl.semaphore_wait(barrier_sem, 1)

        for d in range(N_DEV):
            @pl.when(my != d)
            def _entry():
                pl.semaphore_signal(
                    entry_sems.at[my], inc=1,
                    device_id=(d,), device_id_type=pl.DeviceIdType.MESH,
                )

        for m in range(N_DEV):
            @pl.when(my == m)
            def _dev():
                for c in range(N_DEV):
                    j = (m + 1 + c) % N_DEV
                    y_c = jnp.maximum(
                        jnp.dot(x_ref[...], w_ref[:, j * n_per:(j + 1) * n_per],
                                preferred_element_type=jnp.float32),
                        0.0,
                    )
                    if j == m:
                        out_ref[m * m_per:(m + 1) * m_per, :] = y_c
                    else:
                        ybuf[c] = y_c.astype(jnp.bfloat16)
                        pl.semaphore_wait(entry_sems.at[j], 1)
                        pltpu.make_async_remote_copy(
                            src_ref=ybuf.at[c],
                            dst_ref=rbuf.at[m],
                            send_sem=send_sems.at[c],
                            recv_sem=recv_sems.at[m],
                            device_id=(j,),
                            device_id_type=pl.DeviceIdType.MESH,
                        ).start()

        for c in range(N_DEV - 1):
            pltpu.make_async_remote_copy(
                src_ref=ybuf.at[c],
                dst_ref=rbuf.at[0],
                send_sem=send_sems.at[c],
                recv_sem=recv_sems.at[0],
                device_id=(0,),
                device_id_type=pl.DeviceIdType.MESH,
            ).wait_send()

        for m in range(N_DEV):
            @pl.when(my == m)
            def _recv():
                for c in range(N_DEV - 1):
                    src = (m - 1 - c) % N_DEV
                    pltpu.make_async_remote_copy(
                        src_ref=ybuf.at[0],
                        dst_ref=rbuf.at[src],
                        send_sem=send_sems.at[0],
                        recv_sem=recv_sems.at[src],
                        device_id=(src,),
                        device_id_type=pl.DeviceIdType.MESH,
                    ).wait_recv()
                    out_ref[src * m_per:(src + 1) * m_per, :] = (
                        rbuf[src].astype(jnp.float32))

    return pl.pallas_call(
        body,
        out_shape=jax.ShapeDtypeStruct((N_DEV * m_per, n_per), jnp.float32),
        in_specs=[
            pl.BlockSpec(memory_space=pltpu.MemorySpace.VMEM),
            pl.BlockSpec(memory_space=pltpu.MemorySpace.VMEM),
        ],
        out_specs=pl.BlockSpec(memory_space=pltpu.MemorySpace.VMEM),
        scratch_shapes=[
            pltpu.VMEM((N_DEV - 1, m_per, n_per), jnp.bfloat16),
            pltpu.VMEM((N_DEV, m_per, n_per), jnp.bfloat16),
            pltpu.SemaphoreType.DMA((N_DEV - 1,)),
            pltpu.SemaphoreType.DMA((N_DEV,)),
            pltpu.SemaphoreType.REGULAR((N_DEV,)),
        ],
        compiler_params=pltpu.CompilerParams(collective_id=0),
    )(x, w_mat)
